# Initial kernel scaffold; baseline (speedup 1.0000x reference)
import jax
import jax.numpy as jnp
from jax.experimental import pallas as pl


def kernel(x, edge_index, batch, W1, b1, W2, b2, W3, b3, fW1, fb1, fW2, fb2):
    raise NotImplementedError("write your pallas kernel here")


# speedup vs baseline: 5.5007x; 5.5007x over previous
import functools

import jax
import jax.numpy as jnp
from jax import lax
from jax.experimental import pallas as pl
from jax.experimental.pallas import tpu as pltpu
from jax.experimental.pallas import tpu_sc as plsc

G = 2000
NEG = -3.0e38

NSC = 25000
ACC = 25088
STRIPE = 1568
TRASH = 25024
CH = 128

_mesh = functools.partial(
    plsc.VectorSubcoreMesh, core_axis_name="c", subcore_axis_name="s")


def _wid():
    return lax.axis_index("c"), lax.axis_index("s")


def _localize(draw_ref, dloc_ref, base, limit, trash):
    for k in range(CH // 16):
        d = draw_ref[pl.ds(k * 16, 16)]
        l = d - base
        ok = (l >= 0) & (l < limit)
        dloc_ref[pl.ds(k * 16, 16)] = jnp.where(ok, l, trash)


def _copy_out_stripes(acc_ref, out_ref, bounce, cid, sid):
    last = NSC - 15 * STRIPE

    @pl.when(sid < 15)
    def _():
        pltpu.sync_copy(acc_ref.at[pl.ds(sid * STRIPE, STRIPE)], bounce)
        pltpu.sync_copy(bounce,
                        out_ref.at[pl.ds(cid * NSC + sid * STRIPE, STRIPE)])

    @pl.when(sid == 15)
    def _():
        pltpu.sync_copy(acc_ref.at[pl.ds(15 * STRIPE, last)],
                        bounce.at[pl.ds(0, last)])
        pltpu.sync_copy(bounce.at[pl.ds(0, last)],
                        out_ref.at[pl.ds(cid * NSC + 15 * STRIPE, last)])


def _hist_sc(dst2_ref, batch2_ref, deg_ref, cnt_ref,
             acc_deg, acc_cnt, zbuf, ones, draw, dloc, dummy):
    cid, sid = _wid()
    ec = dst2_ref.shape[0] // 16
    bc = batch2_ref.shape[0] // 16

    for i in range(STRIPE // 16):
        zbuf[pl.ds(i * 16, 16)] = jnp.zeros((16,), jnp.float32)
    for i in range(CH // 16):
        ones[pl.ds(i * 16, 16)] = jnp.ones((16,), jnp.float32)
    pltpu.sync_copy(zbuf, acc_deg.at[pl.ds(sid * STRIPE, STRIPE)])
    pltpu.sync_copy(zbuf.at[pl.ds(0, 64)], acc_cnt.at[pl.ds(sid * 64, 64)])
    plsc.subcore_barrier()

    def edge_body(c, carry):
        pltpu.sync_copy(dst2_ref.at[sid * ec + c], draw)
        _localize(draw, dloc, cid * NSC, NSC, TRASH)
        pltpu.sync_copy(ones, acc_deg.at[dloc], add=True)
        return carry
    lax.fori_loop(0, ec, edge_body, 0)

    def batch_body(c, carry):
        pltpu.sync_copy(batch2_ref.at[sid * bc + c], draw)
        _localize(draw, dloc, cid * 1000, 1000, 1016)
        pltpu.sync_copy(ones, acc_cnt.at[dloc], add=True)
        return carry
    lax.fori_loop(0, bc, batch_body, 0)
    plsc.subcore_barrier()

    _copy_out_stripes(acc_deg, deg_ref, zbuf, cid, sid)

    @pl.when(sid < 15)
    def _():
        pltpu.sync_copy(acc_cnt.at[pl.ds(sid * 64, 64)], zbuf.at[pl.ds(0, 64)])
        pltpu.sync_copy(zbuf.at[pl.ds(0, 64)],
                        cnt_ref.at[pl.ds(cid * 1000 + sid * 64, 64)])

    @pl.when(sid == 15)
    def _():
        pltpu.sync_copy(acc_cnt.at[pl.ds(960, 40)], zbuf.at[pl.ds(0, 40)])
        pltpu.sync_copy(zbuf.at[pl.ds(0, 40)],
                        cnt_ref.at[pl.ds(cid * 1000 + 960, 40)])


def _hist_call(dst2, batch2, n):
    return pl.kernel(
        _hist_sc,
        out_type=(jax.ShapeDtypeStruct((n,), jnp.float32),
                  jax.ShapeDtypeStruct((G,), jnp.float32)),
        mesh=_mesh(),
        compiler_params=pltpu.CompilerParams(use_tc_tiling_on_sc=False, needs_layout_passes=False),
        scratch_types=[
            pltpu.VMEM_SHARED((ACC,), jnp.float32),
            pltpu.VMEM_SHARED((1024,), jnp.float32),
            pltpu.VMEM((STRIPE,), jnp.float32),
            pltpu.VMEM((CH,), jnp.float32),
            pltpu.VMEM((CH,), jnp.int32),
            pltpu.VMEM((CH,), jnp.int32),
            pltpu.SemaphoreType.DMA,
        ],
    )(dst2, batch2)


def _agg_sc(u_ref, src2_ref, dst2_ref, y_ref,
            acc, sbuf, draw, dloc, rows, bounce, sem):
    cid, sid = _wid()
    ec = src2_ref.shape[0] // 16
    last = NSC - 15 * STRIPE

    @pl.when(sid < 15)
    def _():
        pltpu.sync_copy(u_ref.at[pl.ds(cid * NSC + sid * STRIPE, STRIPE)],
                        bounce)
        pltpu.sync_copy(bounce, acc.at[pl.ds(sid * STRIPE, STRIPE)])

    @pl.when(sid == 15)
    def _():
        pltpu.sync_copy(u_ref.at[pl.ds(cid * NSC + 15 * STRIPE, last)],
                        bounce.at[pl.ds(0, last)])
        pltpu.sync_copy(bounce.at[pl.ds(0, last)],
                        acc.at[pl.ds(15 * STRIPE, last)])
    plsc.subcore_barrier()

    def edge_body(c, carry):
        pltpu.sync_copy(src2_ref.at[sid * ec + c], sbuf)
        pltpu.sync_copy(dst2_ref.at[sid * ec + c], draw)
        pltpu.async_copy(u_ref.at[sbuf], rows, sem).wait()
        _localize(draw, dloc, cid * NSC, NSC, TRASH)
        pltpu.sync_copy(rows, acc.at[dloc], add=True)
        return carry
    lax.fori_loop(0, ec, edge_body, 0)
    plsc.subcore_barrier()

    _copy_out_stripes(acc, y_ref, bounce, cid, sid)


def _agg_call(u, src2, dst2):
    n = u.shape[0]
    if u.ndim == 1:
        acc_t = pltpu.VMEM_SHARED((ACC,), jnp.float32)
        rows_t = pltpu.VMEM((CH,), jnp.float32)
        bounce_t = pltpu.VMEM((STRIPE,), jnp.float32)
        out_t = jax.ShapeDtypeStruct((n,), jnp.float32)
    else:
        w = u.shape[1]
        acc_t = pltpu.VMEM_SHARED((ACC, w), jnp.float32)
        rows_t = pltpu.VMEM((CH, w), jnp.float32)
        bounce_t = pltpu.VMEM((STRIPE, w), jnp.float32)
        out_t = jax.ShapeDtypeStruct((n, w), jnp.float32)
    return pl.kernel(
        _agg_sc,
        out_type=out_t,
        mesh=_mesh(),
        compiler_params=pltpu.CompilerParams(use_tc_tiling_on_sc=False, needs_layout_passes=False),
        scratch_types=[
            acc_t,
            pltpu.VMEM((CH,), jnp.int32),
            pltpu.VMEM((CH,), jnp.int32),
            pltpu.VMEM((CH,), jnp.int32),
            rows_t,
            bounce_t,
            pltpu.SemaphoreType.DMA,
        ],
    )(u, src2, dst2)


def _pool_sc(h_ref, starts_ref, out_ref, st_v, rowbuf, outrow, sem):
    cid, sid = _wid()
    wid = cid * 16 + sid
    n = h_ref.shape[0]
    gpt = (G + 31) // 32
    g_lo = jnp.minimum(wid * gpt, G)
    g_hi = jnp.minimum(g_lo + gpt, G)

    pltpu.sync_copy(starts_ref, st_v)

    def graph_body(g, carry):
        iv = g + lax.broadcasted_iota(jnp.int32, (16,), 0)
        sv = plsc.load_gather(st_v, [iv])
        s0 = sv[0]
        s1 = sv[1]
        cnt = s1 - s0
        nch = (cnt + 15) // 16

        def chunk_body(c, accs):
            asum, amax = accs
            r0 = s0 + c * 16
            r0c = jnp.minimum(r0, n - 16)
            pltpu.sync_copy(h_ref.at[pl.ds(r0c, 16)], rowbuf)
            for r in range(16):
                node = r0c + r
                valid = (node >= r0) & (node < s1)
                nsum, nmax = [], []
                for f in range(16):
                    v = rowbuf[r, pl.ds(f * 16, 16)]
                    nsum.append(asum[f] + jnp.where(valid, v, 0.0))
                    nmax.append(jnp.maximum(amax[f], jnp.where(valid, v, NEG)))
                asum, amax = nsum, nmax
            return (asum, amax)

        init = ([jnp.zeros((16,), jnp.float32)] * 16,
                [jnp.full((16,), NEG, jnp.float32)] * 16)
        asum, amax = lax.fori_loop(0, nch, chunk_body, init)

        cnt_vec = jnp.broadcast_to(cnt, (16,)).astype(jnp.float32)
        ok_vec = cnt_vec > 0.0
        inv = jnp.where(
            ok_vec, jnp.ones((16,), jnp.float32) / jnp.maximum(cnt_vec, 1.0), 0.0)
        zero16 = jnp.zeros((16,), jnp.float32)
        for f in range(16):
            outrow[pl.ds(f * 16, 16)] = asum[f] * inv
            outrow[pl.ds(256 + f * 16, 16)] = jnp.where(ok_vec, amax[f], zero16)
        pltpu.sync_copy(outrow, out_ref.at[g])
        return carry

    lax.fori_loop(g_lo, g_hi, graph_body, 0)


def _pool_call(h3, starts):
    return pl.kernel(
        _pool_sc,
        out_type=jax.ShapeDtypeStruct((G, 512), jnp.float32),
        mesh=_mesh(),
        compiler_params=pltpu.CompilerParams(use_tc_tiling_on_sc=False, needs_layout_passes=False),
        scratch_types=[
            pltpu.VMEM((starts.shape[0],), jnp.int32),
            pltpu.VMEM((16, 256), jnp.float32),
            pltpu.VMEM((512,), jnp.float32),
            pltpu.SemaphoreType.DMA,
        ],
    )(h3, starts)


def _t0_tc(deg_ref, x_ref, dis_ref, u0_ref):
    deg = deg_ref[...] + 1.0
    dis = lax.rsqrt(deg)
    dis_ref[...] = dis
    u0_ref[...] = dis * x_ref[...]


def _t0_call(deg, x):
    n = deg.shape[0]
    blk = 2000
    grid = n // blk
    bs = pl.BlockSpec((blk, 1), lambda i: (i, 0))
    return pl.pallas_call(
        _t0_tc,
        grid=(grid,),
        in_specs=[bs, bs],
        out_specs=[bs, bs],
        out_shape=[jax.ShapeDtypeStruct((n, 1), jnp.float32),
                   jax.ShapeDtypeStruct((n, 1), jnp.float32)],
    )(deg, x)


def _starts_tc(cnt_ref, out_ref):
    c = cnt_ref[...]
    i = lax.broadcasted_iota(jnp.int32, (125, 125), 0)
    j = lax.broadcasted_iota(jnp.int32, (125, 125), 1)
    lt = jnp.where(i > j, 1.0, 0.0)
    rs = jnp.sum(c, axis=1, keepdims=True)
    rex = jnp.dot(lt, rs, preferred_element_type=jnp.float32, precision=lax.Precision.HIGHEST)
    i2 = lax.broadcasted_iota(jnp.int32, (16, 16), 0)
    j2 = lax.broadcasted_iota(jnp.int32, (16, 16), 1)
    lt2 = jnp.where(i2 < j2, 1.0, 0.0)
    wex = jnp.dot(c, lt2, preferred_element_type=jnp.float32, precision=lax.Precision.HIGHEST)
    out_ref[...] = (rex + wex).astype(jnp.int32)


def _starts_call(cnt, n):
    st = pl.pallas_call(
        _starts_tc,
        out_shape=jax.ShapeDtypeStruct((125, 16), jnp.int32),
    )(cnt.reshape(125, 16))
    return jnp.concatenate(
        [st.reshape(2000), jnp.full((8,), n, jnp.int32)])


def _t1_tc(agg0_ref, dis_ref, w1_ref, b1_ref, ua_ref, ub_ref):
    dis = dis_ref[...]
    z = dis * agg0_ref[...]
    h = jax.nn.relu(z * w1_ref[...] + b1_ref[...])
    u = dis * h
    ua_ref[...] = u[:, 0:32]
    ub_ref[...] = u[:, 32:64]


def _t1_call(agg0, dis, W1, b1):
    n = agg0.shape[0]
    blk = 2000
    bs1 = pl.BlockSpec((blk, 1), lambda i: (i, 0))
    bw = pl.BlockSpec((1, 64), lambda i: (0, 0))
    bo = pl.BlockSpec((blk, 32), lambda i: (i, 0))
    return pl.pallas_call(
        _t1_tc,
        grid=(n // blk,),
        in_specs=[bs1, bs1, bw, bw],
        out_specs=[bo, bo],
        out_shape=[jax.ShapeDtypeStruct((n, 32), jnp.float32),
                   jax.ShapeDtypeStruct((n, 32), jnp.float32)],
    )(agg0, dis, W1, b1)


def _t2_tc(agga_ref, aggb_ref, dis_ref, w2_ref, b2_ref,
           ua_ref, ub_ref, uc_ref, ud_ref):
    dis = dis_ref[...]
    z = dis * jnp.concatenate([agga_ref[...], aggb_ref[...]], axis=1)
    h = jax.nn.relu(
        jnp.dot(z, w2_ref[...], preferred_element_type=jnp.float32, precision=lax.Precision.HIGHEST) + b2_ref[...])
    u = dis * h
    ua_ref[...] = u[:, 0:32]
    ub_ref[...] = u[:, 32:64]
    uc_ref[...] = u[:, 64:96]
    ud_ref[...] = u[:, 96:128]


def _t2_call(agga, aggb, dis, W2, b2):
    n = agga.shape[0]
    blk = 2000
    bi = pl.BlockSpec((blk, 32), lambda i: (i, 0))
    return pl.pallas_call(
        _t2_tc,
        grid=(n // blk,),
        in_specs=[bi, bi,
                  pl.BlockSpec((blk, 1), lambda i: (i, 0)),
                  pl.BlockSpec((64, 128), lambda i: (0, 0)),
                  pl.BlockSpec((1, 128), lambda i: (0, 0))],
        out_specs=[bi, bi, bi, bi],
        out_shape=[jax.ShapeDtypeStruct((n, 32), jnp.float32)] * 4,
    )(agga, aggb, dis, W2, b2)


def _t3_tc(ya_ref, yb_ref, yc_ref, yd_ref, dis_ref, w3_ref, b3_ref, h3_ref):
    dis = dis_ref[...]
    z = dis * jnp.concatenate(
        [ya_ref[...], yb_ref[...], yc_ref[...], yd_ref[...]], axis=1)
    h3_ref[...] = (
        jnp.dot(z, w3_ref[...], preferred_element_type=jnp.float32, precision=lax.Precision.HIGHEST) + b3_ref[...])


def _t3_call(ys, dis, W3, b3):
    n = ys[0].shape[0]
    blk = 2000
    bi = pl.BlockSpec((blk, 32), lambda i: (i, 0))
    return pl.pallas_call(
        _t3_tc,
        grid=(n // blk,),
        in_specs=[bi, bi, bi, bi,
                  pl.BlockSpec((blk, 1), lambda i: (i, 0)),
                  pl.BlockSpec((128, 256), lambda i: (0, 0)),
                  pl.BlockSpec((1, 256), lambda i: (0, 0))],
        out_specs=pl.BlockSpec((blk, 256), lambda i: (i, 0)),
        out_shape=jax.ShapeDtypeStruct((n, 256), jnp.float32),
    )(*ys, dis, W3, b3)


def _mlp_tc(p_ref, w1_ref, b1_ref, w2_ref, b2_ref, o_ref):
    z = jax.nn.relu(
        jnp.dot(p_ref[...], w1_ref[...], preferred_element_type=jnp.float32, precision=lax.Precision.HIGHEST)
        + b1_ref[...])
    o_ref[...] = (
        jnp.dot(z, w2_ref[...], preferred_element_type=jnp.float32, precision=lax.Precision.HIGHEST) + b2_ref[...])


def _mlp_call(pooled, fW1, fb1, fW2, fb2):
    return pl.pallas_call(
        _mlp_tc,
        out_shape=jax.ShapeDtypeStruct((G, 12), jnp.float32),
    )(pooled, fW1, fb1, fW2, fb2)


def kernel(x, edge_index, batch, W1, b1, W2, b2, W3, b3, fW1, fb1, fW2, fb2):
    n = x.shape[0]
    e = edge_index.shape[1]

    src = edge_index[0]
    dst = edge_index[1]

    epc = 32 * CH
    ep = ((e + epc - 1) // epc) * epc
    src2 = jnp.concatenate(
        [src, jnp.zeros((ep - e,), jnp.int32)]).reshape(ep // CH, CH)
    dst2 = jnp.concatenate(
        [dst, jnp.full((ep - e,), n, jnp.int32)]).reshape(ep // CH, CH)
    np_ = ((n + epc - 1) // epc) * epc
    batch2 = jnp.concatenate(
        [batch, jnp.full((np_ - n,), G, jnp.int32)]).reshape(np_ // CH, CH)

    deg, cnt = _hist_call(dst2, batch2, n)
    dis, u0 = _t0_call(deg.reshape(n, 1), x)
    starts = _starts_call(cnt, n)

    agg0 = _agg_call(u0.reshape(n), src2, dst2)
    u1a, u1b = _t1_call(agg0.reshape(n, 1), dis, W1, b1.reshape(1, 64))

    agg1a = _agg_call(u1a, src2, dst2)
    agg1b = _agg_call(u1b, src2, dst2)
    u2 = _t2_call(agg1a, agg1b, dis, W2, b2.reshape(1, 128))

    y3 = [_agg_call(u, src2, dst2) for u in u2]
    h3 = _t3_call(y3, dis, W3, b3.reshape(1, 256))

    pooled = _pool_call(h3, starts)
    out = _mlp_call(pooled, fW1, fb1.reshape(1, 128), fW2, fb2.reshape(1, 12))
    return out


# speedup vs baseline: 7.5444x; 1.3715x over previous
import functools

import jax
import jax.numpy as jnp
from jax import lax
from jax.experimental import pallas as pl
from jax.experimental.pallas import tpu as pltpu
from jax.experimental.pallas import tpu_sc as plsc

G = 2000
NEG = -3.0e38

NSC = 25000
ACC = 25088
STRIPE = 1568
TRASH = 25024
CH = 128
BR = 7
BLK = BR * CH
BCH = 392

_mesh = functools.partial(
    plsc.VectorSubcoreMesh, core_axis_name="c", subcore_axis_name="s")


def _wid():
    return lax.axis_index("c"), lax.axis_index("s")


def _localize(draw_ref, dloc_ref, base, limit, trash):
    for k in range(CH // 16):
        d = draw_ref[pl.ds(k * 16, 16)]
        l = d - base
        ok = (l >= 0) & (l < limit)
        dloc_ref[pl.ds(k * 16, 16)] = jnp.where(ok, l, trash)


def _stripe_chunks(sid_is_last):
    if not sid_is_last:
        return [(q * BCH, BCH) for q in range(STRIPE // BCH)]
    last = NSC - 15 * STRIPE
    full = last // BCH
    out = [(q * BCH, BCH) for q in range(full)]
    if last % BCH:
        out.append((full * BCH, last % BCH))
    return out


def _acc_hbm_copy(acc_ref, hbm_ref, bounce, cid, sid, to_hbm):
    for is_last in (False, True):
        @pl.when((sid == 15) if is_last else (sid < 15))
        def _():
            for off, sz in _stripe_chunks(is_last):
                a = acc_ref.at[pl.ds(sid * STRIPE + off, sz)]
                h = hbm_ref.at[pl.ds(cid * NSC + sid * STRIPE + off, sz)]
                b = bounce.at[pl.ds(0, sz)]
                if to_hbm:
                    pltpu.sync_copy(a, b)
                    pltpu.sync_copy(b, h)
                else:
                    pltpu.sync_copy(h, b)
                    pltpu.sync_copy(b, a)


def _hist_sc(dst2_ref, batch2_ref, deg_ref, cnt_ref,
             acc_deg, acc_cnt, zbuf, ones, draw, dloc, dummy):
    cid, sid = _wid()
    ec = dst2_ref.shape[0] // 16
    bc = batch2_ref.shape[0] // 16

    for i in range(STRIPE // 16):
        zbuf[pl.ds(i * 16, 16)] = jnp.zeros((16,), jnp.float32)
    for i in range(CH // 16):
        ones[pl.ds(i * 16, 16)] = jnp.ones((16,), jnp.float32)
    pltpu.sync_copy(zbuf, acc_deg.at[pl.ds(sid * STRIPE, STRIPE)])
    pltpu.sync_copy(zbuf.at[pl.ds(0, 64)], acc_cnt.at[pl.ds(sid * 64, 64)])
    plsc.subcore_barrier()

    def edge_body(c, carry):
        pltpu.sync_copy(dst2_ref.at[sid * ec + c], draw)
        _localize(draw, dloc, cid * NSC, NSC, TRASH)
        pltpu.sync_copy(ones, acc_deg.at[dloc], add=True)
        return carry
    lax.fori_loop(0, ec, edge_body, 0)

    def batch_body(c, carry):
        pltpu.sync_copy(batch2_ref.at[sid * bc + c], draw)
        _localize(draw, dloc, cid * 1000, 1000, 1016)
        pltpu.sync_copy(ones, acc_cnt.at[dloc], add=True)
        return carry
    lax.fori_loop(0, bc, batch_body, 0)
    plsc.subcore_barrier()

    _acc_hbm_copy(acc_deg, deg_ref, zbuf.at[pl.ds(0, BCH)], cid, sid, to_hbm=True)

    @pl.when(sid < 15)
    def _():
        pltpu.sync_copy(acc_cnt.at[pl.ds(sid * 64, 64)], zbuf.at[pl.ds(0, 64)])
        pltpu.sync_copy(zbuf.at[pl.ds(0, 64)],
                        cnt_ref.at[pl.ds(cid * 1000 + sid * 64, 64)])

    @pl.when(sid == 15)
    def _():
        pltpu.sync_copy(acc_cnt.at[pl.ds(960, 40)], zbuf.at[pl.ds(0, 40)])
        pltpu.sync_copy(zbuf.at[pl.ds(0, 40)],
                        cnt_ref.at[pl.ds(cid * 1000 + 960, 40)])


def _hist_call(dst2, batch2, n):
    return pl.kernel(
        _hist_sc,
        out_type=(jax.ShapeDtypeStruct((n,), jnp.float32),
                  jax.ShapeDtypeStruct((G,), jnp.float32)),
        mesh=_mesh(),
        compiler_params=pltpu.CompilerParams(use_tc_tiling_on_sc=False, needs_layout_passes=False),
        scratch_types=[
            pltpu.VMEM_SHARED((ACC,), jnp.float32),
            pltpu.VMEM_SHARED((1024,), jnp.float32),
            pltpu.VMEM((STRIPE,), jnp.float32),
            pltpu.VMEM((CH,), jnp.float32),
            pltpu.VMEM((CH,), jnp.int32),
            pltpu.VMEM((CH,), jnp.int32),
            pltpu.SemaphoreType.DMA,
        ],
    )(dst2, batch2)


def _agg_sc(u_ref, src2_ref, dst2_ref, y_ref,
            acc, sb0, db0, dl0, rw0, sb1, db1, dl1, rw1, bounce, sg0, sg1):
    cid, sid = _wid()
    _acc_hbm_copy(acc, u_ref, bounce, cid, sid, to_hbm=False)
    plsc.subcore_barrier()

    rpt = src2_ref.shape[0] // 16
    nb = rpt // BR
    pairs = nb // 2
    base_row = sid * rpt
    bufs = ((sb0, db0, dl0, rw0, sg0), (sb1, db1, dl1, rw1, sg1))

    def prefetch(b, p):
        sb, db, dl, rw, sg = bufs[p]
        r0 = base_row + b * BR
        pltpu.sync_copy(src2_ref.at[pl.ds(r0, BR)], sb)
        pltpu.sync_copy(dst2_ref.at[pl.ds(r0, BR)], db)
        for r in range(BR):
            for k in range(CH // 16):
                d = db[r, pl.ds(k * 16, 16)]
                l = d - cid * NSC
                ok = (l >= 0) & (l < NSC)
                dl[r, pl.ds(k * 16, 16)] = jnp.where(ok, l, TRASH)
        for r in range(BR):
            pltpu.async_copy(u_ref.at[sb.at[r]], rw.at[r], sg)

    def consume(p):
        sb, db, dl, rw, sg = bufs[p]
        for r in range(BR):
            pltpu.make_async_copy(u_ref.at[sb.at[r]], rw.at[r], sg).wait()
        for r in range(BR):
            pltpu.sync_copy(rw.at[r], acc.at[dl.at[r]], add=True)

    prefetch(0, 0)

    def pair_body(i, carry):
        prefetch(2 * i + 1, 1)
        consume(0)

        @pl.when(i < pairs - 1)
        def _():
            prefetch(2 * i + 2, 0)
        consume(1)
        return carry
    lax.fori_loop(0, pairs, pair_body, 0)
    plsc.subcore_barrier()

    _acc_hbm_copy(acc, y_ref, bounce, cid, sid, to_hbm=True)


def _agg_call(u, src2, dst2):
    n, w = u.shape
    idx_t = pltpu.VMEM((BR, CH), jnp.int32)
    rows_t = pltpu.VMEM((BR, CH, w), jnp.float32)
    return pl.kernel(
        _agg_sc,
        out_type=jax.ShapeDtypeStruct((n, w), jnp.float32),
        mesh=_mesh(),
        compiler_params=pltpu.CompilerParams(use_tc_tiling_on_sc=False, needs_layout_passes=False),
        scratch_types=[
            pltpu.VMEM_SHARED((ACC, w), jnp.float32),
            idx_t, idx_t, idx_t, rows_t,
            idx_t, idx_t, idx_t, rows_t,
            pltpu.VMEM((BCH, w), jnp.float32),
            pltpu.SemaphoreType.DMA,
            pltpu.SemaphoreType.DMA,
        ],
    )(u, src2, dst2)


def _pool_sc(h_ref, starts_ref, out_ref, st_v, rowbuf, outrow, sem):
    cid, sid = _wid()
    wid = cid * 16 + sid
    n = h_ref.shape[0]
    gpt = (G + 31) // 32
    g_lo = jnp.minimum(wid * gpt, G)
    g_hi = jnp.minimum(g_lo + gpt, G)

    pltpu.sync_copy(starts_ref, st_v)

    def graph_body(g, carry):
        iv = g + lax.broadcasted_iota(jnp.int32, (16,), 0)
        sv = plsc.load_gather(st_v, [iv])
        s0 = sv[0]
        s1 = sv[1]
        cnt = s1 - s0
        nch = (cnt + 15) // 16

        def chunk_body(c, accs):
            asum, amax = accs
            r0 = s0 + c * 16
            r0c = jnp.minimum(r0, n - 16)
            pltpu.sync_copy(h_ref.at[pl.ds(r0c, 16)], rowbuf)
            for r in range(16):
                node = r0c + r
                valid = (node >= r0) & (node < s1)
                nsum, nmax = [], []
                for f in range(16):
                    v = rowbuf[r, pl.ds(f * 16, 16)]
                    nsum.append(asum[f] + jnp.where(valid, v, 0.0))
                    nmax.append(jnp.maximum(amax[f], jnp.where(valid, v, NEG)))
                asum, amax = nsum, nmax
            return (asum, amax)

        init = ([jnp.zeros((16,), jnp.float32)] * 16,
                [jnp.full((16,), NEG, jnp.float32)] * 16)
        asum, amax = lax.fori_loop(0, nch, chunk_body, init)

        cnt_vec = jnp.broadcast_to(cnt, (16,)).astype(jnp.float32)
        ok_vec = cnt_vec > 0.0
        inv = jnp.where(
            ok_vec, jnp.ones((16,), jnp.float32) / jnp.maximum(cnt_vec, 1.0), 0.0)
        zero16 = jnp.zeros((16,), jnp.float32)
        for f in range(16):
            outrow[pl.ds(f * 16, 16)] = asum[f] * inv
            outrow[pl.ds(256 + f * 16, 16)] = jnp.where(ok_vec, amax[f], zero16)
        pltpu.sync_copy(outrow, out_ref.at[g])
        return carry

    lax.fori_loop(g_lo, g_hi, graph_body, 0)


def _pool_call(h3, starts):
    return pl.kernel(
        _pool_sc,
        out_type=jax.ShapeDtypeStruct((G, 512), jnp.float32),
        mesh=_mesh(),
        compiler_params=pltpu.CompilerParams(use_tc_tiling_on_sc=False, needs_layout_passes=False),
        scratch_types=[
            pltpu.VMEM((starts.shape[0],), jnp.int32),
            pltpu.VMEM((16, 256), jnp.float32),
            pltpu.VMEM((512,), jnp.float32),
            pltpu.SemaphoreType.DMA,
        ],
    )(h3, starts)


def _t0_tc(deg_ref, x_ref, dis_ref, u0_ref):
    deg = deg_ref[...] + 1.0
    dis = lax.rsqrt(deg)
    dis_ref[...] = dis
    u0_ref[...] = dis * x_ref[...]


def _t0_call(deg, x):
    n = deg.shape[0]
    blk = 2000
    grid = n // blk
    bs = pl.BlockSpec((blk, 1), lambda i: (i, 0))
    return pl.pallas_call(
        _t0_tc,
        grid=(grid,),
        in_specs=[bs, bs],
        out_specs=[bs, bs],
        out_shape=[jax.ShapeDtypeStruct((n, 1), jnp.float32),
                   jax.ShapeDtypeStruct((n, 1), jnp.float32)],
    )(deg, x)


def _starts_tc(cnt_ref, out_ref):
    c = cnt_ref[...]
    i = lax.broadcasted_iota(jnp.int32, (125, 125), 0)
    j = lax.broadcasted_iota(jnp.int32, (125, 125), 1)
    lt = jnp.where(i > j, 1.0, 0.0)
    rs = jnp.sum(c, axis=1, keepdims=True)
    rex = jnp.dot(lt, rs, preferred_element_type=jnp.float32, precision=lax.Precision.HIGHEST)
    i2 = lax.broadcasted_iota(jnp.int32, (16, 16), 0)
    j2 = lax.broadcasted_iota(jnp.int32, (16, 16), 1)
    lt2 = jnp.where(i2 < j2, 1.0, 0.0)
    wex = jnp.dot(c, lt2, preferred_element_type=jnp.float32, precision=lax.Precision.HIGHEST)
    out_ref[...] = (rex + wex).astype(jnp.int32)


def _starts_call(cnt, n):
    st = pl.pallas_call(
        _starts_tc,
        out_shape=jax.ShapeDtypeStruct((125, 16), jnp.int32),
    )(cnt.reshape(125, 16))
    return jnp.concatenate(
        [st.reshape(2000), jnp.full((8,), n, jnp.int32)])


def _t1_tc(agg0_ref, dis_ref, w1_ref, b1_ref, ua_ref, ub_ref):
    dis = dis_ref[...]
    z = dis * agg0_ref[...]
    h = jax.nn.relu(z * w1_ref[...] + b1_ref[...])
    u = dis * h
    ua_ref[...] = u[:, 0:32]
    ub_ref[...] = u[:, 32:64]


def _t1_call(agg0, dis, W1, b1):
    n = agg0.shape[0]
    blk = 2000
    bs1 = pl.BlockSpec((blk, 1), lambda i: (i, 0))
    bw = pl.BlockSpec((1, 64), lambda i: (0, 0))
    bo = pl.BlockSpec((blk, 32), lambda i: (i, 0))
    return pl.pallas_call(
        _t1_tc,
        grid=(n // blk,),
        in_specs=[bs1, bs1, bw, bw],
        out_specs=[bo, bo],
        out_shape=[jax.ShapeDtypeStruct((n, 32), jnp.float32),
                   jax.ShapeDtypeStruct((n, 32), jnp.float32)],
    )(agg0, dis, W1, b1)


def _t2_tc(agga_ref, aggb_ref, dis_ref, w2_ref, b2_ref,
           ua_ref, ub_ref, uc_ref, ud_ref):
    dis = dis_ref[...]
    z = dis * jnp.concatenate([agga_ref[...], aggb_ref[...]], axis=1)
    h = jax.nn.relu(
        jnp.dot(z, w2_ref[...], preferred_element_type=jnp.float32, precision=lax.Precision.HIGHEST) + b2_ref[...])
    u = dis * h
    ua_ref[...] = u[:, 0:32]
    ub_ref[...] = u[:, 32:64]
    uc_ref[...] = u[:, 64:96]
    ud_ref[...] = u[:, 96:128]


def _t2_call(agga, aggb, dis, W2, b2):
    n = agga.shape[0]
    blk = 2000
    bi = pl.BlockSpec((blk, 32), lambda i: (i, 0))
    return pl.pallas_call(
        _t2_tc,
        grid=(n // blk,),
        in_specs=[bi, bi,
                  pl.BlockSpec((blk, 1), lambda i: (i, 0)),
                  pl.BlockSpec((64, 128), lambda i: (0, 0)),
                  pl.BlockSpec((1, 128), lambda i: (0, 0))],
        out_specs=[bi, bi, bi, bi],
        out_shape=[jax.ShapeDtypeStruct((n, 32), jnp.float32)] * 4,
    )(agga, aggb, dis, W2, b2)


def _t3_tc(ya_ref, yb_ref, yc_ref, yd_ref, dis_ref, w3_ref, b3_ref, h3_ref):
    dis = dis_ref[...]
    z = dis * jnp.concatenate(
        [ya_ref[...], yb_ref[...], yc_ref[...], yd_ref[...]], axis=1)
    h3_ref[...] = (
        jnp.dot(z, w3_ref[...], preferred_element_type=jnp.float32, precision=lax.Precision.HIGHEST) + b3_ref[...])


def _t3_call(ys, dis, W3, b3):
    n = ys[0].shape[0]
    blk = 2000
    bi = pl.BlockSpec((blk, 32), lambda i: (i, 0))
    return pl.pallas_call(
        _t3_tc,
        grid=(n // blk,),
        in_specs=[bi, bi, bi, bi,
                  pl.BlockSpec((blk, 1), lambda i: (i, 0)),
                  pl.BlockSpec((128, 256), lambda i: (0, 0)),
                  pl.BlockSpec((1, 256), lambda i: (0, 0))],
        out_specs=pl.BlockSpec((blk, 256), lambda i: (i, 0)),
        out_shape=jax.ShapeDtypeStruct((n, 256), jnp.float32),
    )(*ys, dis, W3, b3)


def _mlp_tc(p_ref, w1_ref, b1_ref, w2_ref, b2_ref, o_ref):
    z = jax.nn.relu(
        jnp.dot(p_ref[...], w1_ref[...], preferred_element_type=jnp.float32, precision=lax.Precision.HIGHEST)
        + b1_ref[...])
    o_ref[...] = (
        jnp.dot(z, w2_ref[...], preferred_element_type=jnp.float32, precision=lax.Precision.HIGHEST) + b2_ref[...])


def _mlp_call(pooled, fW1, fb1, fW2, fb2):
    return pl.pallas_call(
        _mlp_tc,
        out_shape=jax.ShapeDtypeStruct((G, 12), jnp.float32),
    )(pooled, fW1, fb1, fW2, fb2)


def kernel(x, edge_index, batch, W1, b1, W2, b2, W3, b3, fW1, fb1, fW2, fb2):
    n = x.shape[0]
    e = edge_index.shape[1]

    src = edge_index[0]
    dst = edge_index[1]

    epc = 32 * CH
    ep = ((e + epc - 1) // epc) * epc
    src2 = jnp.concatenate(
        [src, jnp.zeros((ep - e,), jnp.int32)]).reshape(ep // CH, CH)
    dst2 = jnp.concatenate(
        [dst, jnp.full((ep - e,), n, jnp.int32)]).reshape(ep // CH, CH)
    np_ = ((n + epc - 1) // epc) * epc
    batch2 = jnp.concatenate(
        [batch, jnp.full((np_ - n,), G, jnp.int32)]).reshape(np_ // CH, CH)

    deg, cnt = _hist_call(dst2, batch2, n)
    dis, u0 = _t0_call(deg.reshape(n, 1), x)
    starts = _starts_call(cnt, n)

    agg0 = _agg_call(u0, src2, dst2)
    u1a, u1b = _t1_call(agg0, dis, W1, b1.reshape(1, 64))

    agg1a = _agg_call(u1a, src2, dst2)
    agg1b = _agg_call(u1b, src2, dst2)
    u2 = _t2_call(agg1a, agg1b, dis, W2, b2.reshape(1, 128))

    y3 = [_agg_call(u, src2, dst2) for u in u2]
    h3 = _t3_call(y3, dis, W3, b3.reshape(1, 256))

    pooled = _pool_call(h3, starts)
    out = _mlp_call(pooled, fW1, fb1.reshape(1, 128), fW2, fb2.reshape(1, 12))
    return out


# speedup vs baseline: 7.5925x; 1.0064x over previous
import functools

import jax
import jax.numpy as jnp
from jax import lax
from jax.experimental import pallas as pl
from jax.experimental.pallas import tpu as pltpu
from jax.experimental.pallas import tpu_sc as plsc

G = 2000
NEG = -3.0e38

NSC = 25000
ACC = 25088
STRIPE = 1568
TRASH = 25024
CH = 128
BR = 7
BLK = BR * CH
BCH = 392

_mesh = functools.partial(
    plsc.VectorSubcoreMesh, core_axis_name="c", subcore_axis_name="s")


def _wid():
    return lax.axis_index("c"), lax.axis_index("s")


def _localize(draw_ref, dloc_ref, base, limit, trash):
    for k in range(CH // 16):
        d = draw_ref[pl.ds(k * 16, 16)]
        l = d - base
        ok = (l >= 0) & (l < limit)
        dloc_ref[pl.ds(k * 16, 16)] = jnp.where(ok, l, trash)


def _stripe_chunks(sid_is_last):
    if not sid_is_last:
        return [(q * BCH, BCH) for q in range(STRIPE // BCH)]
    last = NSC - 15 * STRIPE
    full = last // BCH
    out = [(q * BCH, BCH) for q in range(full)]
    if last % BCH:
        out.append((full * BCH, last % BCH))
    return out


def _acc_hbm_copy(acc_ref, hbm_ref, bounce, cid, sid, to_hbm):
    for is_last in (False, True):
        @pl.when((sid == 15) if is_last else (sid < 15))
        def _():
            for off, sz in _stripe_chunks(is_last):
                a = acc_ref.at[pl.ds(sid * STRIPE + off, sz)]
                h = hbm_ref.at[pl.ds(cid * NSC + sid * STRIPE + off, sz)]
                b = bounce.at[pl.ds(0, sz)]
                if to_hbm:
                    pltpu.sync_copy(a, b)
                    pltpu.sync_copy(b, h)
                else:
                    pltpu.sync_copy(h, b)
                    pltpu.sync_copy(b, a)


def _hist_sc(dst2_ref, batch2_ref, deg_ref, cnt_ref,
             acc_deg, acc_cnt, zbuf, ones, draw, dloc, dummy):
    cid, sid = _wid()
    ec = dst2_ref.shape[0] // 16
    bc = batch2_ref.shape[0] // 16

    for i in range(STRIPE // 16):
        zbuf[pl.ds(i * 16, 16)] = jnp.zeros((16,), jnp.float32)
    for i in range(CH // 16):
        ones[pl.ds(i * 16, 16)] = jnp.ones((16,), jnp.float32)
    pltpu.sync_copy(zbuf, acc_deg.at[pl.ds(sid * STRIPE, STRIPE)])
    pltpu.sync_copy(zbuf.at[pl.ds(0, 64)], acc_cnt.at[pl.ds(sid * 64, 64)])
    plsc.subcore_barrier()

    def edge_body(c, carry):
        pltpu.sync_copy(dst2_ref.at[sid * ec + c], draw)
        _localize(draw, dloc, cid * NSC, NSC, TRASH)
        pltpu.sync_copy(ones, acc_deg.at[dloc], add=True)
        return carry
    lax.fori_loop(0, ec, edge_body, 0)

    def batch_body(c, carry):
        pltpu.sync_copy(batch2_ref.at[sid * bc + c], draw)
        _localize(draw, dloc, cid * 1000, 1000, 1016)
        pltpu.sync_copy(ones, acc_cnt.at[dloc], add=True)
        return carry
    lax.fori_loop(0, bc, batch_body, 0)
    plsc.subcore_barrier()

    _acc_hbm_copy(acc_deg, deg_ref, zbuf.at[pl.ds(0, BCH)], cid, sid, to_hbm=True)

    @pl.when(sid < 15)
    def _():
        pltpu.sync_copy(acc_cnt.at[pl.ds(sid * 64, 64)], zbuf.at[pl.ds(0, 64)])
        pltpu.sync_copy(zbuf.at[pl.ds(0, 64)],
                        cnt_ref.at[pl.ds(cid * 1000 + sid * 64, 64)])

    @pl.when(sid == 15)
    def _():
        pltpu.sync_copy(acc_cnt.at[pl.ds(960, 40)], zbuf.at[pl.ds(0, 40)])
        pltpu.sync_copy(zbuf.at[pl.ds(0, 40)],
                        cnt_ref.at[pl.ds(cid * 1000 + 960, 40)])


def _hist_call(dst2, batch2, n):
    return pl.kernel(
        _hist_sc,
        out_type=(jax.ShapeDtypeStruct((n,), jnp.float32),
                  jax.ShapeDtypeStruct((G,), jnp.float32)),
        mesh=_mesh(),
        compiler_params=pltpu.CompilerParams(use_tc_tiling_on_sc=False, needs_layout_passes=False),
        scratch_types=[
            pltpu.VMEM_SHARED((ACC,), jnp.float32),
            pltpu.VMEM_SHARED((1024,), jnp.float32),
            pltpu.VMEM((STRIPE,), jnp.float32),
            pltpu.VMEM((CH,), jnp.float32),
            pltpu.VMEM((CH,), jnp.int32),
            pltpu.VMEM((CH,), jnp.int32),
            pltpu.SemaphoreType.DMA,
        ],
    )(dst2, batch2)


def _agg_sc(u_ref, src2_ref, dst2_ref, y_ref,
            acc, sb0, db0, dl0, rw0, sb1, db1, dl1, rw1, bounce, sg0, sg1):
    cid, sid = _wid()
    _acc_hbm_copy(acc, u_ref, bounce, cid, sid, to_hbm=False)
    plsc.subcore_barrier()

    rpt = src2_ref.shape[0] // 16
    nb = rpt // BR
    pairs = nb // 2
    base_row = sid * rpt
    bufs = ((sb0, db0, dl0, rw0, sg0), (sb1, db1, dl1, rw1, sg1))

    def prefetch(b, p):
        sb, db, dl, rw, sg = bufs[p]
        r0 = base_row + b * BR
        pltpu.sync_copy(src2_ref.at[pl.ds(r0, BR)], sb)
        pltpu.sync_copy(dst2_ref.at[pl.ds(r0, BR)], db)
        for r in range(BR):
            for k in range(CH // 16):
                d = db[r, pl.ds(k * 16, 16)]
                l = d - cid * NSC
                ok = (l >= 0) & (l < NSC)
                dl[r, pl.ds(k * 16, 16)] = jnp.where(ok, l, TRASH)
        for r in range(BR):
            pltpu.async_copy(u_ref.at[sb.at[r]], rw.at[r], sg)

    def consume(p):
        sb, db, dl, rw, sg = bufs[p]
        for r in range(BR):
            pltpu.make_async_copy(u_ref.at[sb.at[r]], rw.at[r], sg).wait()
        for r in range(BR):
            pltpu.sync_copy(rw.at[r], acc.at[dl.at[r]], add=True)

    prefetch(0, 0)

    def pair_body(i, carry):
        prefetch(2 * i + 1, 1)
        consume(0)

        @pl.when(i < pairs - 1)
        def _():
            prefetch(2 * i + 2, 0)
        consume(1)
        return carry
    lax.fori_loop(0, pairs, pair_body, 0)
    plsc.subcore_barrier()

    _acc_hbm_copy(acc, y_ref, bounce, cid, sid, to_hbm=True)


def _agg_call(u, src2, dst2):
    n, w = u.shape
    idx_t = pltpu.VMEM((BR, CH), jnp.int32)
    if w == 1:
        out = pl.kernel(
            _agg_sc,
            out_type=jax.ShapeDtypeStruct((n,), jnp.float32),
            mesh=_mesh(),
            compiler_params=pltpu.CompilerParams(
                use_tc_tiling_on_sc=False, needs_layout_passes=False),
            scratch_types=[
                pltpu.VMEM_SHARED((ACC,), jnp.float32),
                idx_t, idx_t, idx_t, pltpu.VMEM((BR, CH), jnp.float32),
                idx_t, idx_t, idx_t, pltpu.VMEM((BR, CH), jnp.float32),
                pltpu.VMEM((BCH,), jnp.float32),
                pltpu.SemaphoreType.DMA,
                pltpu.SemaphoreType.DMA,
            ],
        )(u.reshape(n), src2, dst2)
        return out.reshape(n, 1)
    rows_t = pltpu.VMEM((BR, CH, w), jnp.float32)
    return pl.kernel(
        _agg_sc,
        out_type=jax.ShapeDtypeStruct((n, w), jnp.float32),
        mesh=_mesh(),
        compiler_params=pltpu.CompilerParams(use_tc_tiling_on_sc=False, needs_layout_passes=False),
        scratch_types=[
            pltpu.VMEM_SHARED((ACC, w), jnp.float32),
            idx_t, idx_t, idx_t, rows_t,
            idx_t, idx_t, idx_t, rows_t,
            pltpu.VMEM((BCH, w), jnp.float32),
            pltpu.SemaphoreType.DMA,
            pltpu.SemaphoreType.DMA,
        ],
    )(u, src2, dst2)


def _pool_sc(h_ref, starts_ref, out_ref, st_v, rowbuf, outrow, sem):
    cid, sid = _wid()
    wid = cid * 16 + sid
    n = h_ref.shape[0]
    gpt = (G + 31) // 32
    g_lo = jnp.minimum(wid * gpt, G)
    g_hi = jnp.minimum(g_lo + gpt, G)

    pltpu.sync_copy(starts_ref, st_v)

    def graph_body(g, carry):
        iv = g + lax.broadcasted_iota(jnp.int32, (16,), 0)
        sv = plsc.load_gather(st_v, [iv])
        s0 = sv[0]
        s1 = sv[1]
        cnt = s1 - s0
        nch = (cnt + 15) // 16

        def chunk_body(c, accs):
            asum, amax = accs
            r0 = s0 + c * 16
            r0c = jnp.minimum(r0, n - 16)
            pltpu.sync_copy(h_ref.at[pl.ds(r0c, 16)], rowbuf)
            for r in range(16):
                node = r0c + r
                valid = (node >= r0) & (node < s1)
                nsum, nmax = [], []
                for f in range(16):
                    v = rowbuf[r, pl.ds(f * 16, 16)]
                    nsum.append(asum[f] + jnp.where(valid, v, 0.0))
                    nmax.append(jnp.maximum(amax[f], jnp.where(valid, v, NEG)))
                asum, amax = nsum, nmax
            return (asum, amax)

        init = ([jnp.zeros((16,), jnp.float32)] * 16,
                [jnp.full((16,), NEG, jnp.float32)] * 16)
        asum, amax = lax.fori_loop(0, nch, chunk_body, init)

        cnt_vec = jnp.broadcast_to(cnt, (16,)).astype(jnp.float32)
        ok_vec = cnt_vec > 0.0
        inv = jnp.where(
            ok_vec, jnp.ones((16,), jnp.float32) / jnp.maximum(cnt_vec, 1.0), 0.0)
        zero16 = jnp.zeros((16,), jnp.float32)
        for f in range(16):
            outrow[pl.ds(f * 16, 16)] = asum[f] * inv
            outrow[pl.ds(256 + f * 16, 16)] = jnp.where(ok_vec, amax[f], zero16)
        pltpu.sync_copy(outrow, out_ref.at[g])
        return carry

    lax.fori_loop(g_lo, g_hi, graph_body, 0)


def _pool_call(h3, starts):
    return pl.kernel(
        _pool_sc,
        out_type=jax.ShapeDtypeStruct((G, 512), jnp.float32),
        mesh=_mesh(),
        compiler_params=pltpu.CompilerParams(use_tc_tiling_on_sc=False, needs_layout_passes=False),
        scratch_types=[
            pltpu.VMEM((starts.shape[0],), jnp.int32),
            pltpu.VMEM((16, 256), jnp.float32),
            pltpu.VMEM((512,), jnp.float32),
            pltpu.SemaphoreType.DMA,
        ],
    )(h3, starts)


def _t0_tc(deg_ref, x_ref, dis_ref, u0_ref):
    deg = deg_ref[...] + 1.0
    dis = lax.rsqrt(deg)
    dis_ref[...] = dis
    u0_ref[...] = dis * x_ref[...]


def _t0_call(deg, x):
    n = deg.shape[0]
    blk = 2000
    grid = n // blk
    bs = pl.BlockSpec((blk, 1), lambda i: (i, 0))
    return pl.pallas_call(
        _t0_tc,
        grid=(grid,),
        in_specs=[bs, bs],
        out_specs=[bs, bs],
        out_shape=[jax.ShapeDtypeStruct((n, 1), jnp.float32),
                   jax.ShapeDtypeStruct((n, 1), jnp.float32)],
    )(deg, x)


def _starts_tc(cnt_ref, out_ref):
    c = cnt_ref[...]
    i = lax.broadcasted_iota(jnp.int32, (125, 125), 0)
    j = lax.broadcasted_iota(jnp.int32, (125, 125), 1)
    lt = jnp.where(i > j, 1.0, 0.0)
    rs = jnp.sum(c, axis=1, keepdims=True)
    rex = jnp.dot(lt, rs, preferred_element_type=jnp.float32, precision=lax.Precision.HIGHEST)
    i2 = lax.broadcasted_iota(jnp.int32, (16, 16), 0)
    j2 = lax.broadcasted_iota(jnp.int32, (16, 16), 1)
    lt2 = jnp.where(i2 < j2, 1.0, 0.0)
    wex = jnp.dot(c, lt2, preferred_element_type=jnp.float32, precision=lax.Precision.HIGHEST)
    out_ref[...] = (rex + wex).astype(jnp.int32)


def _starts_call(cnt, n):
    st = pl.pallas_call(
        _starts_tc,
        out_shape=jax.ShapeDtypeStruct((125, 16), jnp.int32),
    )(cnt.reshape(125, 16))
    return jnp.concatenate(
        [st.reshape(2000), jnp.full((8,), n, jnp.int32)])


def _t1_tc(agg0_ref, dis_ref, w1_ref, b1_ref, ua_ref, ub_ref):
    dis = dis_ref[...]
    z = dis * agg0_ref[...]
    h = jax.nn.relu(z * w1_ref[...] + b1_ref[...])
    u = dis * h
    ua_ref[...] = u[:, 0:32]
    ub_ref[...] = u[:, 32:64]


def _t1_call(agg0, dis, W1, b1):
    n = agg0.shape[0]
    blk = 2000
    bs1 = pl.BlockSpec((blk, 1), lambda i: (i, 0))
    bw = pl.BlockSpec((1, 64), lambda i: (0, 0))
    bo = pl.BlockSpec((blk, 32), lambda i: (i, 0))
    return pl.pallas_call(
        _t1_tc,
        grid=(n // blk,),
        in_specs=[bs1, bs1, bw, bw],
        out_specs=[bo, bo],
        out_shape=[jax.ShapeDtypeStruct((n, 32), jnp.float32),
                   jax.ShapeDtypeStruct((n, 32), jnp.float32)],
    )(agg0, dis, W1, b1)


def _t2_tc(agga_ref, aggb_ref, dis_ref, w2_ref, b2_ref,
           ua_ref, ub_ref, uc_ref, ud_ref):
    dis = dis_ref[...]
    z = dis * jnp.concatenate([agga_ref[...], aggb_ref[...]], axis=1)
    h = jax.nn.relu(
        jnp.dot(z, w2_ref[...], preferred_element_type=jnp.float32, precision=lax.Precision.HIGHEST) + b2_ref[...])
    u = dis * h
    ua_ref[...] = u[:, 0:32]
    ub_ref[...] = u[:, 32:64]
    uc_ref[...] = u[:, 64:96]
    ud_ref[...] = u[:, 96:128]


def _t2_call(agga, aggb, dis, W2, b2):
    n = agga.shape[0]
    blk = 2000
    bi = pl.BlockSpec((blk, 32), lambda i: (i, 0))
    return pl.pallas_call(
        _t2_tc,
        grid=(n // blk,),
        in_specs=[bi, bi,
                  pl.BlockSpec((blk, 1), lambda i: (i, 0)),
                  pl.BlockSpec((64, 128), lambda i: (0, 0)),
                  pl.BlockSpec((1, 128), lambda i: (0, 0))],
        out_specs=[bi, bi, bi, bi],
        out_shape=[jax.ShapeDtypeStruct((n, 32), jnp.float32)] * 4,
    )(agga, aggb, dis, W2, b2)


def _t3_tc(ya_ref, yb_ref, yc_ref, yd_ref, dis_ref, w3_ref, b3_ref, h3_ref):
    dis = dis_ref[...]
    z = dis * jnp.concatenate(
        [ya_ref[...], yb_ref[...], yc_ref[...], yd_ref[...]], axis=1)
    h3_ref[...] = (
        jnp.dot(z, w3_ref[...], preferred_element_type=jnp.float32, precision=lax.Precision.HIGHEST) + b3_ref[...])


def _t3_call(ys, dis, W3, b3):
    n = ys[0].shape[0]
    blk = 2000
    bi = pl.BlockSpec((blk, 32), lambda i: (i, 0))
    return pl.pallas_call(
        _t3_tc,
        grid=(n // blk,),
        in_specs=[bi, bi, bi, bi,
                  pl.BlockSpec((blk, 1), lambda i: (i, 0)),
                  pl.BlockSpec((128, 256), lambda i: (0, 0)),
                  pl.BlockSpec((1, 256), lambda i: (0, 0))],
        out_specs=pl.BlockSpec((blk, 256), lambda i: (i, 0)),
        out_shape=jax.ShapeDtypeStruct((n, 256), jnp.float32),
    )(*ys, dis, W3, b3)


def _mlp_tc(p_ref, w1_ref, b1_ref, w2_ref, b2_ref, o_ref):
    z = jax.nn.relu(
        jnp.dot(p_ref[...], w1_ref[...], preferred_element_type=jnp.float32, precision=lax.Precision.HIGHEST)
        + b1_ref[...])
    o_ref[...] = (
        jnp.dot(z, w2_ref[...], preferred_element_type=jnp.float32, precision=lax.Precision.HIGHEST) + b2_ref[...])


def _mlp_call(pooled, fW1, fb1, fW2, fb2):
    return pl.pallas_call(
        _mlp_tc,
        out_shape=jax.ShapeDtypeStruct((G, 12), jnp.float32),
    )(pooled, fW1, fb1, fW2, fb2)


def kernel(x, edge_index, batch, W1, b1, W2, b2, W3, b3, fW1, fb1, fW2, fb2):
    n = x.shape[0]
    e = edge_index.shape[1]

    src = edge_index[0]
    dst = edge_index[1]

    epc = 32 * CH
    ep = ((e + epc - 1) // epc) * epc
    src2 = jnp.concatenate(
        [src, jnp.zeros((ep - e,), jnp.int32)]).reshape(ep // CH, CH)
    dst2 = jnp.concatenate(
        [dst, jnp.full((ep - e,), n, jnp.int32)]).reshape(ep // CH, CH)
    np_ = ((n + epc - 1) // epc) * epc
    batch2 = jnp.concatenate(
        [batch, jnp.full((np_ - n,), G, jnp.int32)]).reshape(np_ // CH, CH)

    deg, cnt = _hist_call(dst2, batch2, n)
    dis, u0 = _t0_call(deg.reshape(n, 1), x)
    starts = _starts_call(cnt, n)

    agg0 = _agg_call(u0, src2, dst2)
    u1a, u1b = _t1_call(agg0, dis, W1, b1.reshape(1, 64))

    agg1a = _agg_call(u1a, src2, dst2)
    agg1b = _agg_call(u1b, src2, dst2)
    u2 = _t2_call(agg1a, agg1b, dis, W2, b2.reshape(1, 128))

    y3 = [_agg_call(u, src2, dst2) for u in u2]
    h3 = _t3_call(y3, dis, W3, b3.reshape(1, 256))

    pooled = _pool_call(h3, starts)
    out = _mlp_call(pooled, fW1, fb1.reshape(1, 128), fW2, fb2.reshape(1, 12))
    return out


# speedup vs baseline: 7.5946x; 1.0003x over previous
import functools

import jax
import jax.numpy as jnp
from jax import lax
from jax.experimental import pallas as pl
from jax.experimental.pallas import tpu as pltpu
from jax.experimental.pallas import tpu_sc as plsc

G = 2000
NEG = -3.0e38

NSC = 25000
ACC = 25088
STRIPE = 1568
TRASH = 25024
CH = 128
BR = 7
BLK = BR * CH
BCH = 392

_mesh = functools.partial(
    plsc.VectorSubcoreMesh, core_axis_name="c", subcore_axis_name="s")


def _wid():
    return lax.axis_index("c"), lax.axis_index("s")


def _localize(draw_ref, dloc_ref, base, limit, trash):
    for k in range(CH // 16):
        d = draw_ref[pl.ds(k * 16, 16)]
        l = d - base
        ok = (l >= 0) & (l < limit)
        dloc_ref[pl.ds(k * 16, 16)] = jnp.where(ok, l, trash)


def _stripe_chunks(sid_is_last):
    if not sid_is_last:
        return [(q * BCH, BCH) for q in range(STRIPE // BCH)]
    last = NSC - 15 * STRIPE
    full = last // BCH
    out = [(q * BCH, BCH) for q in range(full)]
    if last % BCH:
        out.append((full * BCH, last % BCH))
    return out


def _acc_hbm_copy(acc_ref, hbm_ref, bounce, cid, sid, to_hbm):
    for is_last in (False, True):
        @pl.when((sid == 15) if is_last else (sid < 15))
        def _():
            for off, sz in _stripe_chunks(is_last):
                a = acc_ref.at[pl.ds(sid * STRIPE + off, sz)]
                h = hbm_ref.at[pl.ds(cid * NSC + sid * STRIPE + off, sz)]
                b = bounce.at[pl.ds(0, sz)]
                if to_hbm:
                    pltpu.sync_copy(a, b)
                    pltpu.sync_copy(b, h)
                else:
                    pltpu.sync_copy(h, b)
                    pltpu.sync_copy(b, a)


def _hist_sc(dst2_ref, batch2_ref, deg_ref, cnt_ref,
             acc_deg, acc_cnt, zbuf, ones, draw, dloc, dummy):
    cid, sid = _wid()
    ec = dst2_ref.shape[0] // 16
    bc = batch2_ref.shape[0] // 16

    for i in range(STRIPE // 16):
        zbuf[pl.ds(i * 16, 16)] = jnp.zeros((16,), jnp.float32)
    for i in range(CH // 16):
        ones[pl.ds(i * 16, 16)] = jnp.ones((16,), jnp.float32)
    pltpu.sync_copy(zbuf, acc_deg.at[pl.ds(sid * STRIPE, STRIPE)])
    pltpu.sync_copy(zbuf.at[pl.ds(0, 64)], acc_cnt.at[pl.ds(sid * 64, 64)])
    plsc.subcore_barrier()

    def scan_blocked(arr_ref, acc_ref, rows, base, limit, trash):
        nb2 = (arr_ref.shape[0] // 16) // rows

        def body(b, carry):
            r0 = sid * (arr_ref.shape[0] // 16) + b * rows
            pltpu.sync_copy(arr_ref.at[pl.ds(r0, rows)],
                            draw.at[pl.ds(0, rows)])
            for r in range(rows):
                for k in range(CH // 16):
                    d = draw[r, pl.ds(k * 16, 16)]
                    l = d - base
                    ok = (l >= 0) & (l < limit)
                    dloc[r, pl.ds(k * 16, 16)] = jnp.where(ok, l, trash)
            for r in range(rows):
                pltpu.sync_copy(ones, acc_ref.at[dloc.at[r]], add=True)
            return carry
        lax.fori_loop(0, nb2, body, 0)

    scan_blocked(dst2_ref, acc_deg, BR, cid * NSC, NSC, TRASH)
    scan_blocked(batch2_ref, acc_cnt, 13, cid * 1000, 1000, 1016)
    plsc.subcore_barrier()

    _acc_hbm_copy(acc_deg, deg_ref, zbuf.at[pl.ds(0, BCH)], cid, sid, to_hbm=True)

    @pl.when(sid < 15)
    def _():
        pltpu.sync_copy(acc_cnt.at[pl.ds(sid * 64, 64)], zbuf.at[pl.ds(0, 64)])
        pltpu.sync_copy(zbuf.at[pl.ds(0, 64)],
                        cnt_ref.at[pl.ds(cid * 1000 + sid * 64, 64)])

    @pl.when(sid == 15)
    def _():
        pltpu.sync_copy(acc_cnt.at[pl.ds(960, 40)], zbuf.at[pl.ds(0, 40)])
        pltpu.sync_copy(zbuf.at[pl.ds(0, 40)],
                        cnt_ref.at[pl.ds(cid * 1000 + 960, 40)])


def _hist_call(dst2, batch2, n):
    return pl.kernel(
        _hist_sc,
        out_type=(jax.ShapeDtypeStruct((n,), jnp.float32),
                  jax.ShapeDtypeStruct((G,), jnp.float32)),
        mesh=_mesh(),
        compiler_params=pltpu.CompilerParams(use_tc_tiling_on_sc=False, needs_layout_passes=False),
        scratch_types=[
            pltpu.VMEM_SHARED((ACC,), jnp.float32),
            pltpu.VMEM_SHARED((1024,), jnp.float32),
            pltpu.VMEM((STRIPE,), jnp.float32),
            pltpu.VMEM((CH,), jnp.float32),
            pltpu.VMEM((13, CH), jnp.int32),
            pltpu.VMEM((13, CH), jnp.int32),
            pltpu.SemaphoreType.DMA,
        ],
    )(dst2, batch2)


def _agg_sc(u_ref, src2_ref, dst2_ref, y_ref,
            acc, sb0, db0, dl0, rw0, sb1, db1, dl1, rw1, bounce, sg0, sg1):
    cid, sid = _wid()
    _acc_hbm_copy(acc, u_ref, bounce, cid, sid, to_hbm=False)
    plsc.subcore_barrier()

    rpt = src2_ref.shape[0] // 16
    nb = rpt // BR
    pairs = nb // 2
    base_row = sid * rpt
    bufs = ((sb0, db0, dl0, rw0, sg0), (sb1, db1, dl1, rw1, sg1))

    def prefetch(b, p):
        sb, db, dl, rw, sg = bufs[p]
        r0 = base_row + b * BR
        pltpu.sync_copy(src2_ref.at[pl.ds(r0, BR)], sb)
        pltpu.sync_copy(dst2_ref.at[pl.ds(r0, BR)], db)
        for r in range(BR):
            for k in range(CH // 16):
                d = db[r, pl.ds(k * 16, 16)]
                l = d - cid * NSC
                ok = (l >= 0) & (l < NSC)
                dl[r, pl.ds(k * 16, 16)] = jnp.where(ok, l, TRASH)
        for r in range(BR):
            pltpu.async_copy(u_ref.at[sb.at[r]], rw.at[r], sg)

    def consume(p):
        sb, db, dl, rw, sg = bufs[p]
        for r in range(BR):
            pltpu.make_async_copy(u_ref.at[sb.at[r]], rw.at[r], sg).wait()
        for r in range(BR):
            pltpu.sync_copy(rw.at[r], acc.at[dl.at[r]], add=True)

    prefetch(0, 0)

    def pair_body(i, carry):
        prefetch(2 * i + 1, 1)
        consume(0)

        @pl.when(i < pairs - 1)
        def _():
            prefetch(2 * i + 2, 0)
        consume(1)
        return carry
    lax.fori_loop(0, pairs, pair_body, 0)
    plsc.subcore_barrier()

    _acc_hbm_copy(acc, y_ref, bounce, cid, sid, to_hbm=True)


def _agg_call(u, src2, dst2):
    n, w = u.shape
    idx_t = pltpu.VMEM((BR, CH), jnp.int32)
    if w == 1:
        out = pl.kernel(
            _agg_sc,
            out_type=jax.ShapeDtypeStruct((n,), jnp.float32),
            mesh=_mesh(),
            compiler_params=pltpu.CompilerParams(
                use_tc_tiling_on_sc=False, needs_layout_passes=False),
            scratch_types=[
                pltpu.VMEM_SHARED((ACC,), jnp.float32),
                idx_t, idx_t, idx_t, pltpu.VMEM((BR, CH), jnp.float32),
                idx_t, idx_t, idx_t, pltpu.VMEM((BR, CH), jnp.float32),
                pltpu.VMEM((BCH,), jnp.float32),
                pltpu.SemaphoreType.DMA,
                pltpu.SemaphoreType.DMA,
            ],
        )(u.reshape(n), src2, dst2)
        return out.reshape(n, 1)
    rows_t = pltpu.VMEM((BR, CH, w), jnp.float32)
    return pl.kernel(
        _agg_sc,
        out_type=jax.ShapeDtypeStruct((n, w), jnp.float32),
        mesh=_mesh(),
        compiler_params=pltpu.CompilerParams(use_tc_tiling_on_sc=False, needs_layout_passes=False),
        scratch_types=[
            pltpu.VMEM_SHARED((ACC, w), jnp.float32),
            idx_t, idx_t, idx_t, rows_t,
            idx_t, idx_t, idx_t, rows_t,
            pltpu.VMEM((BCH, w), jnp.float32),
            pltpu.SemaphoreType.DMA,
            pltpu.SemaphoreType.DMA,
        ],
    )(u, src2, dst2)


def _pool_sc(h_ref, starts_ref, out_ref, st_v, rowbuf, outrow, sem):
    cid, sid = _wid()
    wid = cid * 16 + sid
    n = h_ref.shape[0]
    gpt = (G + 31) // 32
    g_lo = jnp.minimum(wid * gpt, G)
    g_hi = jnp.minimum(g_lo + gpt, G)

    pltpu.sync_copy(starts_ref, st_v)

    def graph_body(g, carry):
        iv = g + lax.broadcasted_iota(jnp.int32, (16,), 0)
        sv = plsc.load_gather(st_v, [iv])
        s0 = sv[0]
        s1 = sv[1]
        cnt = s1 - s0
        nch = (cnt + 15) // 16

        def chunk_body(c, accs):
            asum, amax = accs
            r0 = s0 + c * 16
            r0c = jnp.minimum(r0, n - 16)
            pltpu.sync_copy(h_ref.at[pl.ds(r0c, 16)], rowbuf)
            for r in range(16):
                node = r0c + r
                valid = (node >= r0) & (node < s1)
                nsum, nmax = [], []
                for f in range(16):
                    v = rowbuf[r, pl.ds(f * 16, 16)]
                    nsum.append(asum[f] + jnp.where(valid, v, 0.0))
                    nmax.append(jnp.maximum(amax[f], jnp.where(valid, v, NEG)))
                asum, amax = nsum, nmax
            return (asum, amax)

        init = ([jnp.zeros((16,), jnp.float32)] * 16,
                [jnp.full((16,), NEG, jnp.float32)] * 16)
        asum, amax = lax.fori_loop(0, nch, chunk_body, init)

        cnt_vec = jnp.broadcast_to(cnt, (16,)).astype(jnp.float32)
        ok_vec = cnt_vec > 0.0
        inv = jnp.where(
            ok_vec, jnp.ones((16,), jnp.float32) / jnp.maximum(cnt_vec, 1.0), 0.0)
        zero16 = jnp.zeros((16,), jnp.float32)
        for f in range(16):
            outrow[pl.ds(f * 16, 16)] = asum[f] * inv
            outrow[pl.ds(256 + f * 16, 16)] = jnp.where(ok_vec, amax[f], zero16)
        pltpu.sync_copy(outrow, out_ref.at[g])
        return carry

    lax.fori_loop(g_lo, g_hi, graph_body, 0)


def _pool_call(h3, starts):
    return pl.kernel(
        _pool_sc,
        out_type=jax.ShapeDtypeStruct((G, 512), jnp.float32),
        mesh=_mesh(),
        compiler_params=pltpu.CompilerParams(use_tc_tiling_on_sc=False, needs_layout_passes=False),
        scratch_types=[
            pltpu.VMEM((starts.shape[0],), jnp.int32),
            pltpu.VMEM((16, 256), jnp.float32),
            pltpu.VMEM((512,), jnp.float32),
            pltpu.SemaphoreType.DMA,
        ],
    )(h3, starts)


def _t0_tc(deg_ref, x_ref, dis_ref, u0_ref):
    deg = deg_ref[...] + 1.0
    dis = lax.rsqrt(deg)
    dis_ref[...] = dis
    u0_ref[...] = dis * x_ref[...]


def _t0_call(deg, x):
    n = deg.shape[0]
    blk = 2000
    grid = n // blk
    bs = pl.BlockSpec((blk, 1), lambda i: (i, 0))
    return pl.pallas_call(
        _t0_tc,
        grid=(grid,),
        in_specs=[bs, bs],
        out_specs=[bs, bs],
        out_shape=[jax.ShapeDtypeStruct((n, 1), jnp.float32),
                   jax.ShapeDtypeStruct((n, 1), jnp.float32)],
    )(deg, x)


def _starts_tc(cnt_ref, out_ref):
    c = cnt_ref[...]
    i = lax.broadcasted_iota(jnp.int32, (125, 125), 0)
    j = lax.broadcasted_iota(jnp.int32, (125, 125), 1)
    lt = jnp.where(i > j, 1.0, 0.0)
    rs = jnp.sum(c, axis=1, keepdims=True)
    rex = jnp.dot(lt, rs, preferred_element_type=jnp.float32, precision=lax.Precision.HIGHEST)
    i2 = lax.broadcasted_iota(jnp.int32, (16, 16), 0)
    j2 = lax.broadcasted_iota(jnp.int32, (16, 16), 1)
    lt2 = jnp.where(i2 < j2, 1.0, 0.0)
    wex = jnp.dot(c, lt2, preferred_element_type=jnp.float32, precision=lax.Precision.HIGHEST)
    out_ref[...] = (rex + wex).astype(jnp.int32)


def _starts_call(cnt, n):
    st = pl.pallas_call(
        _starts_tc,
        out_shape=jax.ShapeDtypeStruct((125, 16), jnp.int32),
    )(cnt.reshape(125, 16))
    return jnp.concatenate(
        [st.reshape(2000), jnp.full((8,), n, jnp.int32)])


def _t1_tc(agg0_ref, dis_ref, w1_ref, b1_ref, ua_ref, ub_ref):
    dis = dis_ref[...]
    z = dis * agg0_ref[...]
    h = jax.nn.relu(z * w1_ref[...] + b1_ref[...])
    u = dis * h
    ua_ref[...] = u[:, 0:32]
    ub_ref[...] = u[:, 32:64]


def _t1_call(agg0, dis, W1, b1):
    n = agg0.shape[0]
    blk = 2000
    bs1 = pl.BlockSpec((blk, 1), lambda i: (i, 0))
    bw = pl.BlockSpec((1, 64), lambda i: (0, 0))
    bo = pl.BlockSpec((blk, 32), lambda i: (i, 0))
    return pl.pallas_call(
        _t1_tc,
        grid=(n // blk,),
        in_specs=[bs1, bs1, bw, bw],
        out_specs=[bo, bo],
        out_shape=[jax.ShapeDtypeStruct((n, 32), jnp.float32),
                   jax.ShapeDtypeStruct((n, 32), jnp.float32)],
    )(agg0, dis, W1, b1)


def _t2_tc(agga_ref, aggb_ref, dis_ref, w2_ref, b2_ref,
           ua_ref, ub_ref, uc_ref, ud_ref):
    dis = dis_ref[...]
    z = dis * jnp.concatenate([agga_ref[...], aggb_ref[...]], axis=1)
    h = jax.nn.relu(
        jnp.dot(z, w2_ref[...], preferred_element_type=jnp.float32, precision=lax.Precision.HIGHEST) + b2_ref[...])
    u = dis * h
    ua_ref[...] = u[:, 0:32]
    ub_ref[...] = u[:, 32:64]
    uc_ref[...] = u[:, 64:96]
    ud_ref[...] = u[:, 96:128]


def _t2_call(agga, aggb, dis, W2, b2):
    n = agga.shape[0]
    blk = 2000
    bi = pl.BlockSpec((blk, 32), lambda i: (i, 0))
    return pl.pallas_call(
        _t2_tc,
        grid=(n // blk,),
        in_specs=[bi, bi,
                  pl.BlockSpec((blk, 1), lambda i: (i, 0)),
                  pl.BlockSpec((64, 128), lambda i: (0, 0)),
                  pl.BlockSpec((1, 128), lambda i: (0, 0))],
        out_specs=[bi, bi, bi, bi],
        out_shape=[jax.ShapeDtypeStruct((n, 32), jnp.float32)] * 4,
    )(agga, aggb, dis, W2, b2)


def _t3_tc(ya_ref, yb_ref, yc_ref, yd_ref, dis_ref, w3_ref, b3_ref, h3_ref):
    dis = dis_ref[...]
    z = dis * jnp.concatenate(
        [ya_ref[...], yb_ref[...], yc_ref[...], yd_ref[...]], axis=1)
    h3_ref[...] = (
        jnp.dot(z, w3_ref[...], preferred_element_type=jnp.float32, precision=lax.Precision.HIGHEST) + b3_ref[...])


def _t3_call(ys, dis, W3, b3):
    n = ys[0].shape[0]
    blk = 2000
    bi = pl.BlockSpec((blk, 32), lambda i: (i, 0))
    return pl.pallas_call(
        _t3_tc,
        grid=(n // blk,),
        in_specs=[bi, bi, bi, bi,
                  pl.BlockSpec((blk, 1), lambda i: (i, 0)),
                  pl.BlockSpec((128, 256), lambda i: (0, 0)),
                  pl.BlockSpec((1, 256), lambda i: (0, 0))],
        out_specs=pl.BlockSpec((blk, 256), lambda i: (i, 0)),
        out_shape=jax.ShapeDtypeStruct((n, 256), jnp.float32),
    )(*ys, dis, W3, b3)


def _mlp_tc(p_ref, w1_ref, b1_ref, w2_ref, b2_ref, o_ref):
    z = jax.nn.relu(
        jnp.dot(p_ref[...], w1_ref[...], preferred_element_type=jnp.float32, precision=lax.Precision.HIGHEST)
        + b1_ref[...])
    o_ref[...] = (
        jnp.dot(z, w2_ref[...], preferred_element_type=jnp.float32, precision=lax.Precision.HIGHEST) + b2_ref[...])


def _mlp_call(pooled, fW1, fb1, fW2, fb2):
    return pl.pallas_call(
        _mlp_tc,
        out_shape=jax.ShapeDtypeStruct((G, 12), jnp.float32),
    )(pooled, fW1, fb1, fW2, fb2)


def kernel(x, edge_index, batch, W1, b1, W2, b2, W3, b3, fW1, fb1, fW2, fb2):
    n = x.shape[0]
    e = edge_index.shape[1]

    src = edge_index[0]
    dst = edge_index[1]

    epc = 32 * CH
    ep = ((e + epc - 1) // epc) * epc
    src2 = jnp.concatenate(
        [src, jnp.zeros((ep - e,), jnp.int32)]).reshape(ep // CH, CH)
    dst2 = jnp.concatenate(
        [dst, jnp.full((ep - e,), n, jnp.int32)]).reshape(ep // CH, CH)
    np_ = ((n + epc - 1) // epc) * epc
    batch2 = jnp.concatenate(
        [batch, jnp.full((np_ - n,), G, jnp.int32)]).reshape(np_ // CH, CH)

    deg, cnt = _hist_call(dst2, batch2, n)
    dis, u0 = _t0_call(deg.reshape(n, 1), x)
    starts = _starts_call(cnt, n)

    agg0 = _agg_call(u0, src2, dst2)
    u1a, u1b = _t1_call(agg0, dis, W1, b1.reshape(1, 64))

    agg1a = _agg_call(u1a, src2, dst2)
    agg1b = _agg_call(u1b, src2, dst2)
    u2 = _t2_call(agg1a, agg1b, dis, W2, b2.reshape(1, 128))

    y3 = [_agg_call(u, src2, dst2) for u in u2]
    h3 = _t3_call(y3, dis, W3, b3.reshape(1, 256))

    pooled = _pool_call(h3, starts)
    out = _mlp_call(pooled, fW1, fb1.reshape(1, 128), fW2, fb2.reshape(1, 12))
    return out
